# baseline (speedup 1.0000x reference)
import functools

import jax
import jax.numpy as jnp
from jax import lax
from jax.experimental import pallas as pl
from jax.experimental.pallas import tpu as pltpu
from jax.experimental.pallas import tpu_sc as plsc

N = 10000
E = 320000
DIM = 128
K = 15
KP_EXTENT = 0.1 * 1.2 / 2.5
EXT2 = KP_EXTENT * KP_EXTENT
BN_EPS = 1e-5

NC = 2
NS = 16
NW = NC * NS
EPT = E // NW
CH = 2000
NCHUNK = EPT // CH
CGRP = CH // 16
RPT = 632
NPAD = NS * RPT
FGRP = 2
QCAP = 576


def _sqrt16(x):
    i = plsc.bitcast(x, jnp.int32)
    y = plsc.bitcast((i >> 1) + 0x1FBD1DF5, jnp.float32)
    y = 0.5 * (y + x / y)
    y = 0.5 * (y + x / y)
    return y


def _make_sc_kernel():
    mesh = plsc.VectorSubcoreMesh(core_axis_name="c", subcore_axis_name="s")

    @functools.partial(
        pl.kernel,
        out_type=jax.ShapeDtypeStruct((NC, NPAD, DIM), jnp.float32),
        mesh=mesh,
        compiler_params=pltpu.CompilerParams(needs_layout_passes=False),
        scratch_types=[
            pltpu.VMEM_SHARED((NPAD, DIM), jnp.float32),
            pltpu.VMEM((3 * N,), jnp.float32),
            pltpu.VMEM((CH + 16,), jnp.int32),
            pltpu.VMEM((CH + 16,), jnp.int32),
            pltpu.VMEM((CH + 16,), jnp.int32),
            pltpu.VMEM((K * 3 * 128,), jnp.float32),
            pltpu.VMEM((QCAP,), jnp.int32),
            pltpu.VMEM((QCAP,), jnp.int32),
            pltpu.VMEM((QCAP,), jnp.float32),
            pltpu.VMEM((16, DIM), jnp.float32),
            pltpu.VMEM((16, DIM), jnp.float32),
            pltpu.SemaphoreType.DMA,
            pltpu.SemaphoreType.DMA,
        ],
    )
    def sc_kernel(src_hbm, dst_hbm, pos_hbm, z_hbm, kps_hbm, zeros_hbm,
                  out_hbm, acc_sh, pos_v, src_v, dst_v, cand_v,
                  kps_v, qrow_v, qdst_v, qd2_v, rows_a, rows_b,
                  sem_a, sem_b):
        c = lax.axis_index("c")
        s = lax.axis_index("s")
        wid = c * NS + s
        gbase = wid * EPT

        zero16f = jnp.zeros((16,), jnp.float32)
        zero16i = jnp.zeros((16,), jnp.int32)

        pltpu.sync_copy(zeros_hbm.at[pl.ds(s * RPT, RPT)],
                        acc_sh.at[pl.ds(s * RPT, RPT)])

        pltpu.sync_copy(pos_hbm, pos_v)
        pltpu.sync_copy(kps_hbm, kps_v)

        plsc.subcore_barrier()

        bound2 = zero16f
        for k in range(K):
            kx = kps_v[pl.ds((k * 3 + 0) * 128, 16)]
            ky = kps_v[pl.ds((k * 3 + 1) * 128, 16)]
            kz = kps_v[pl.ds((k * 3 + 2) * 128, 16)]
            n2 = kx * kx + ky * ky + kz * kz
            bound2 = jnp.maximum(bound2, 2.0 * (n2 + EXT2))

        iota16 = lax.iota(jnp.int32, 16)
        inv_ext = jnp.float32(1.0 / KP_EXTENT)

        def chunk_body(ci, carry):
            cbase = gbase + ci * CH
            pltpu.sync_copy(src_hbm.at[pl.ds(cbase, CH)],
                            src_v.at[pl.ds(0, CH)])
            pltpu.sync_copy(dst_hbm.at[pl.ds(cbase, CH)],
                            dst_v.at[pl.ds(0, CH)])
            src_v[pl.ds(CH, 16)] = zero16i
            dst_v[pl.ds(CH, 16)] = zero16i

            def stage1(g, ncand):
                base = g * 16
                sv = src_v[pl.ds(base, 16)]
                dv = dst_v[pl.ds(base, 16)]
                sv3 = sv + sv + sv
                dv3 = dv + dv + dv
                sx = plsc.load_gather(pos_v, [sv3])
                sy = plsc.load_gather(pos_v, [sv3 + 1])
                sz = plsc.load_gather(pos_v, [sv3 + 2])
                dx = plsc.load_gather(pos_v, [dv3])
                dy = plsc.load_gather(pos_v, [dv3 + 1])
                dz = plsc.load_gather(pos_v, [dv3 + 2])
                rx = sx - dx
                ry = sy - dy
                rz = sz - dz
                r2 = rx * rx + ry * ry + rz * rz
                m = r2 < bound2
                plsc.store_compressed(cand_v.at[pl.ds(ncand, 16)],
                                      base + iota16, mask=m)
                return ncand + jnp.sum(m.astype(jnp.int32))

            ncand = lax.fori_loop(0, CGRP, stage1, jnp.int32(0))
            cand_v[pl.ds(ncand, 16)] = jnp.full((16,), CH, jnp.int32)

            def append_group(i, qcnt):
                cv = cand_v[pl.ds(i * 16, 16)]
                valid = cv < CH
                sv = plsc.load_gather(src_v, [cv])
                dv = plsc.load_gather(dst_v, [cv])
                sv3 = sv + sv + sv
                dv3 = dv + dv + dv
                sx = plsc.load_gather(pos_v, [sv3])
                sy = plsc.load_gather(pos_v, [sv3 + 1])
                sz = plsc.load_gather(pos_v, [sv3 + 2])
                tx = plsc.load_gather(pos_v, [dv3])
                ty = plsc.load_gather(pos_v, [dv3 + 1])
                tz = plsc.load_gather(pos_v, [dv3 + 2])
                rx = sx - tx
                ry = sy - ty
                rz = sz - tz
                for k in range(K):
                    kx = kps_v[pl.ds((k * 3 + 0) * 128, 16)]
                    ky = kps_v[pl.ds((k * 3 + 1) * 128, 16)]
                    kz = kps_v[pl.ds((k * 3 + 2) * 128, 16)]
                    ddx = rx - kx
                    ddy = ry - ky
                    ddz = rz - kz
                    d2 = ddx * ddx + ddy * ddy + ddz * ddz
                    am = (d2 < EXT2) & valid
                    plsc.store_compressed(qrow_v.at[pl.ds(qcnt, 16)],
                                          sv + (k * N), mask=am)
                    plsc.store_compressed(qdst_v.at[pl.ds(qcnt, 16)], dv,
                                          mask=am)
                    plsc.store_compressed(qd2_v.at[pl.ds(qcnt, 16)],
                                          jnp.maximum(d2, 1e-12), mask=am)
                    qcnt = qcnt + jnp.sum(am.astype(jnp.int32))
                return qcnt

            ncg = (ncand + 15) // 16
            dstpad = jnp.full((16,), N, jnp.int32)
            onesf = jnp.ones((16,), jnp.float32)

            def scale_scatter(b, rows_v):
                dvec = qdst_v[pl.ds(b * 16, 16)]
                d2v = qd2_v[pl.ds(b * 16, 16)]
                w = jnp.maximum(1.0 - _sqrt16(d2v) * inv_ext, 0.0)
                for r in range(16):
                    wb = jnp.full((16,), w[r], jnp.float32)
                    for j in range(DIM // 16):
                        rows_v[r, pl.ds(j * 16, 16)] = (
                            rows_v[r, pl.ds(j * 16, 16)] * wb)
                pltpu.sync_copy(rows_v, acc_sh.at[dvec], add=True)

            def gather_start(b, rows_v, sem):
                pltpu.async_copy(z_hbm.at[qrow_v.at[pl.ds(b * 16, 16)]],
                                 rows_v, sem)

            def gather_wait(b, rows_v, sem):
                pltpu.make_async_copy(
                    z_hbm.at[qrow_v.at[pl.ds(b * 16, 16)]], rows_v,
                    sem).wait()

            def block(o, carry2):
                lo = o * FGRP
                hi = jnp.minimum(lo + FGRP, ncg)
                qcnt = lax.fori_loop(lo, hi, append_group, jnp.int32(0))
                qpad = ((qcnt + 15) // 16) * 16

                def padb(p, inner):
                    qrow_v[pl.ds(qcnt + p * 16, 16)] = zero16i
                    qdst_v[pl.ds(qcnt + p * 16, 16)] = dstpad
                    qd2_v[pl.ds(qcnt + p * 16, 16)] = onesf
                    return inner

                lax.fori_loop(0, (qpad + 48 - qcnt + 15) // 16, padb, 0)
                nb = qpad // 16

                gather_start(0, rows_a, sem_a)

                def fbpair(i, inner):
                    b0 = i * 2
                    gather_wait(b0, rows_a, sem_a)
                    gather_start(b0 + 1, rows_b, sem_b)
                    scale_scatter(b0, rows_a)
                    gather_wait(b0 + 1, rows_b, sem_b)
                    gather_start(b0 + 2, rows_a, sem_a)
                    scale_scatter(b0 + 1, rows_b)
                    return inner

                npairs = (nb + 1) // 2
                lax.fori_loop(0, npairs, fbpair, 0)
                gather_wait(npairs * 2, rows_a, sem_a)
                return carry2

            nblk = (ncg + FGRP - 1) // FGRP
            lax.fori_loop(0, nblk, block, 0)
            return carry

        lax.fori_loop(0, NCHUNK, chunk_body, 0)

        plsc.subcore_barrier()

        pltpu.sync_copy(acc_sh.at[pl.ds(s * RPT, RPT)],
                        out_hbm.at[c].at[pl.ds(s * RPT, RPT)])

    return sc_kernel


_SC_KERNEL = _make_sc_kernel()


def _z_matmul(feats, W):
    BN = 2000

    def body(f_ref, w_ref, o_ref):
        o_ref[...] = jnp.dot(f_ref[...], w_ref[0],
                             preferred_element_type=jnp.float32)[None]

    z = pl.pallas_call(
        body,
        grid=(K, N // BN),
        in_specs=[
            pl.BlockSpec((BN, DIM), lambda k, j: (j, 0)),
            pl.BlockSpec((1, DIM, DIM), lambda k, j: (k, 0, 0)),
        ],
        out_specs=pl.BlockSpec((1, BN, DIM), lambda k, j: (k, j, 0)),
        out_shape=jax.ShapeDtypeStruct((K, N, DIM), jnp.float32),
    )(feats, W)
    return z.reshape(K * N, DIM)


def _bn_act(parts, gamma, beta):
    def body(p_ref, g_ref, b_ref, o_ref):
        x = p_ref[0] + p_ref[1]
        mean = jnp.mean(x, axis=0, keepdims=True)
        xc = x - mean
        var = jnp.mean(xc * xc, axis=0, keepdims=True)
        h = xc * lax.rsqrt(var + BN_EPS) * g_ref[...] + b_ref[...]
        o_ref[...] = jnp.where(h >= 0, h, 0.1 * h)

    return pl.pallas_call(
        body,
        grid=(1,),
        in_specs=[
            pl.BlockSpec((NC, N, DIM), lambda i: (0, 0, 0)),
            pl.BlockSpec((1, DIM), lambda i: (0, 0)),
            pl.BlockSpec((1, DIM), lambda i: (0, 0)),
        ],
        out_specs=pl.BlockSpec((N, DIM), lambda i: (0, 0)),
        out_shape=jax.ShapeDtypeStruct((N, DIM), jnp.float32),
    )(parts, gamma.reshape(1, DIM), beta.reshape(1, DIM))


def kernel(feats, pos, edge_index, KP, W, gamma, beta):
    src = edge_index[0]
    dst = edge_index[1]
    z = _z_matmul(feats, W)
    kps = jnp.broadcast_to(KP[:, :, None], (K, 3, 128))
    kps = kps.astype(jnp.float32).reshape(-1)
    zeros = jnp.zeros((NPAD, DIM), jnp.float32)
    parts = _SC_KERNEL(src, dst, pos.reshape(-1), z, kps, zeros)
    return _bn_act(parts, gamma, beta)


# speedup vs baseline: 1.4404x; 1.4404x over previous
import functools

import jax
import jax.numpy as jnp
from jax import lax
from jax.experimental import pallas as pl
from jax.experimental.pallas import tpu as pltpu
from jax.experimental.pallas import tpu_sc as plsc

N = 10000
E = 320000
DIM = 128
K = 15
KP_EXTENT = 0.1 * 1.2 / 2.5
EXT2 = KP_EXTENT * KP_EXTENT
BN_EPS = 1e-5

NC = 2
NS = 16
NW = NC * NS
EPT = E // NW
CH = 2000
NCHUNK = EPT // CH
CGRP = CH // 16
RPT = 632
NPAD = NS * RPT
QCAP = 256


def _sqrt16(x):
    i = plsc.bitcast(x, jnp.int32)
    y = plsc.bitcast((i >> 1) + 0x1FBD1DF5, jnp.float32)
    y = 0.5 * (y + x / y)
    y = 0.5 * (y + x / y)
    return y


def _make_sc_kernel():
    mesh = plsc.VectorSubcoreMesh(core_axis_name="c", subcore_axis_name="s")

    @functools.partial(
        pl.kernel,
        out_type=jax.ShapeDtypeStruct((NC, NPAD, DIM), jnp.float32),
        mesh=mesh,
        compiler_params=pltpu.CompilerParams(needs_layout_passes=False),
        scratch_types=[
            pltpu.VMEM_SHARED((NPAD, DIM), jnp.float32),
            pltpu.VMEM((3 * N,), jnp.float32),
            pltpu.VMEM((CH + 16,), jnp.int32),
            pltpu.VMEM((CH + 16,), jnp.int32),
            pltpu.VMEM((CH + 16,), jnp.int32),
            pltpu.VMEM((K * 3 * 128,), jnp.float32),
            pltpu.VMEM((QCAP,), jnp.int32),
            pltpu.VMEM((QCAP,), jnp.int32),
            pltpu.VMEM((QCAP,), jnp.float32),
            pltpu.VMEM((16, DIM), jnp.float32),
        ],
    )
    def sc_kernel(src_hbm, dst_hbm, pos_hbm, z_hbm, kps_hbm, zeros_hbm,
                  out_hbm, acc_sh, pos_v, src_v, dst_v, cand_v,
                  kps_v, qrow_v, qdst_v, qd2_v, rows_v):
        c = lax.axis_index("c")
        s = lax.axis_index("s")
        wid = c * NS + s
        gbase = wid * EPT

        zero16f = jnp.zeros((16,), jnp.float32)
        zero16i = jnp.zeros((16,), jnp.int32)

        pltpu.sync_copy(zeros_hbm.at[pl.ds(s * RPT, RPT)],
                        acc_sh.at[pl.ds(s * RPT, RPT)])

        pltpu.sync_copy(pos_hbm, pos_v)
        pltpu.sync_copy(kps_hbm, kps_v)

        plsc.subcore_barrier()

        bound2 = zero16f
        for k in range(K):
            kx = kps_v[pl.ds((k * 3 + 0) * 128, 16)]
            ky = kps_v[pl.ds((k * 3 + 1) * 128, 16)]
            kz = kps_v[pl.ds((k * 3 + 2) * 128, 16)]
            n2 = kx * kx + ky * ky + kz * kz
            bound2 = jnp.maximum(bound2, 2.0 * (n2 + EXT2))

        iota16 = lax.iota(jnp.int32, 16)
        inv_ext = jnp.float32(1.0 / KP_EXTENT)

        def chunk_body(ci, carry):
            cbase = gbase + ci * CH
            pltpu.sync_copy(src_hbm.at[pl.ds(cbase, CH)],
                            src_v.at[pl.ds(0, CH)])
            pltpu.sync_copy(dst_hbm.at[pl.ds(cbase, CH)],
                            dst_v.at[pl.ds(0, CH)])
            src_v[pl.ds(CH, 16)] = zero16i
            dst_v[pl.ds(CH, 16)] = zero16i

            def stage1(g, ncand):
                base = g * 16
                sv = src_v[pl.ds(base, 16)]
                dv = dst_v[pl.ds(base, 16)]
                sv3 = sv + sv + sv
                dv3 = dv + dv + dv
                sx = plsc.load_gather(pos_v, [sv3])
                sy = plsc.load_gather(pos_v, [sv3 + 1])
                sz = plsc.load_gather(pos_v, [sv3 + 2])
                dx = plsc.load_gather(pos_v, [dv3])
                dy = plsc.load_gather(pos_v, [dv3 + 1])
                dz = plsc.load_gather(pos_v, [dv3 + 2])
                rx = sx - dx
                ry = sy - dy
                rz = sz - dz
                r2 = rx * rx + ry * ry + rz * rz
                m = r2 < bound2
                plsc.store_compressed(cand_v.at[pl.ds(ncand, 16)],
                                      base + iota16, mask=m)
                return ncand + jnp.sum(m.astype(jnp.int32))

            ncand = lax.fori_loop(0, CGRP, stage1, jnp.int32(0))
            cand_v[pl.ds(ncand, 16)] = jnp.full((16,), CH, jnp.int32)

            def stage2(i, carry2):
                cv = cand_v[pl.ds(i * 16, 16)]
                valid = cv < CH
                sv = plsc.load_gather(src_v, [cv])
                dv = plsc.load_gather(dst_v, [cv])
                sv3 = sv + sv + sv
                dv3 = dv + dv + dv
                sx = plsc.load_gather(pos_v, [sv3])
                sy = plsc.load_gather(pos_v, [sv3 + 1])
                sz = plsc.load_gather(pos_v, [sv3 + 2])
                tx = plsc.load_gather(pos_v, [dv3])
                ty = plsc.load_gather(pos_v, [dv3 + 1])
                tz = plsc.load_gather(pos_v, [dv3 + 2])
                rx = sx - tx
                ry = sy - ty
                rz = sz - tz
                qcnt = jnp.int32(0)
                for k in range(K):
                    kx = kps_v[pl.ds((k * 3 + 0) * 128, 16)]
                    ky = kps_v[pl.ds((k * 3 + 1) * 128, 16)]
                    kz = kps_v[pl.ds((k * 3 + 2) * 128, 16)]
                    ddx = rx - kx
                    ddy = ry - ky
                    ddz = rz - kz
                    d2 = ddx * ddx + ddy * ddy + ddz * ddz
                    am = (d2 < EXT2) & valid
                    plsc.store_compressed(qrow_v.at[pl.ds(qcnt, 16)],
                                          sv + (k * N), mask=am)
                    plsc.store_compressed(qdst_v.at[pl.ds(qcnt, 16)], dv,
                                          mask=am)
                    plsc.store_compressed(qd2_v.at[pl.ds(qcnt, 16)],
                                          jnp.maximum(d2, 1e-12), mask=am)
                    qcnt = qcnt + jnp.sum(am.astype(jnp.int32))
                qrow_v[pl.ds(qcnt, 16)] = zero16i
                qdst_v[pl.ds(qcnt, 16)] = jnp.full((16,), N, jnp.int32)
                qd2_v[pl.ds(qcnt, 16)] = jnp.ones((16,), jnp.float32)
                nb = (qcnt + 15) // 16

                def fb(b, inner):
                    rvec = qrow_v[pl.ds(b * 16, 16)]
                    dvec = qdst_v[pl.ds(b * 16, 16)]
                    d2v = qd2_v[pl.ds(b * 16, 16)]
                    pltpu.sync_copy(z_hbm.at[rvec], rows_v)
                    w = jnp.maximum(1.0 - _sqrt16(d2v) * inv_ext, 0.0)
                    for r in range(16):
                        wb = jnp.full((16,), w[r], jnp.float32)
                        for j in range(DIM // 16):
                            rows_v[r, pl.ds(j * 16, 16)] = (
                                rows_v[r, pl.ds(j * 16, 16)] * wb)
                    pltpu.sync_copy(rows_v, acc_sh.at[dvec], add=True)
                    return inner

                lax.fori_loop(0, nb, fb, 0)
                return carry2

            ncg = (ncand + 15) // 16
            lax.fori_loop(0, ncg, stage2, 0)
            return carry

        lax.fori_loop(0, NCHUNK, chunk_body, 0)

        plsc.subcore_barrier()

        pltpu.sync_copy(acc_sh.at[pl.ds(s * RPT, RPT)],
                        out_hbm.at[c].at[pl.ds(s * RPT, RPT)])

    return sc_kernel


_SC_KERNEL = _make_sc_kernel()


def _z_matmul(feats, W):
    BN = 2000

    def body(f_ref, w_ref, o_ref):
        o_ref[...] = jnp.dot(f_ref[...], w_ref[0],
                             preferred_element_type=jnp.float32)[None]

    z = pl.pallas_call(
        body,
        grid=(K, N // BN),
        in_specs=[
            pl.BlockSpec((BN, DIM), lambda k, j: (j, 0)),
            pl.BlockSpec((1, DIM, DIM), lambda k, j: (k, 0, 0)),
        ],
        out_specs=pl.BlockSpec((1, BN, DIM), lambda k, j: (k, j, 0)),
        out_shape=jax.ShapeDtypeStruct((K, N, DIM), jnp.float32),
    )(feats, W)
    return z.reshape(K * N, DIM)


def _bn_act(parts, gamma, beta):
    def body(p_ref, g_ref, b_ref, o_ref):
        x = p_ref[0] + p_ref[1]
        mean = jnp.mean(x, axis=0, keepdims=True)
        xc = x - mean
        var = jnp.mean(xc * xc, axis=0, keepdims=True)
        h = xc * lax.rsqrt(var + BN_EPS) * g_ref[...] + b_ref[...]
        o_ref[...] = jnp.where(h >= 0, h, 0.1 * h)

    return pl.pallas_call(
        body,
        grid=(1,),
        in_specs=[
            pl.BlockSpec((NC, N, DIM), lambda i: (0, 0, 0)),
            pl.BlockSpec((1, DIM), lambda i: (0, 0)),
            pl.BlockSpec((1, DIM), lambda i: (0, 0)),
        ],
        out_specs=pl.BlockSpec((N, DIM), lambda i: (0, 0)),
        out_shape=jax.ShapeDtypeStruct((N, DIM), jnp.float32),
    )(parts, gamma.reshape(1, DIM), beta.reshape(1, DIM))


def kernel(feats, pos, edge_index, KP, W, gamma, beta):
    src = edge_index[0]
    dst = edge_index[1]
    z = _z_matmul(feats, W)
    kps = jnp.broadcast_to(KP[:, :, None], (K, 3, 128))
    kps = kps.astype(jnp.float32).reshape(-1)
    zeros = jnp.zeros((NPAD, DIM), jnp.float32)
    parts = _SC_KERNEL(src, dst, pos.reshape(-1), z, kps, zeros)
    return _bn_act(parts, gamma, beta)
